# baseline (speedup 1.0000x reference)
import functools
import math

import jax
import jax.numpy as jnp
from jax import lax
from jax.experimental import pallas as pl
from jax.experimental.pallas import tpu as pltpu
from jax.experimental.pallas import tpu_sc as plsc

B = 16384
D = 32
NC = 2
NS = 16
NW = NC * NS
BPW = B // NW
CHUNK = 128
NCH = BPW // CHUNK

NROWS = 1000000
NTC = NROWS // 128
TAIL = NROWS - NTC * 128
TSTART = NTC * 128
CW = 512
NCW = NTC * 128 // CW
RM_ROWS = NROWS // 4

_mesh = plsc.VectorSubcoreMesh(core_axis_name="c", subcore_axis_name="s")


@functools.partial(
    pl.kernel,
    mesh=_mesh,
    out_type=(
        jax.ShapeDtypeStruct((NROWS * D,), jnp.float32),
        jax.ShapeDtypeStruct((NROWS * D,), jnp.float32),
    ),
    scratch_types=[
        pltpu.VMEM((D, CW), jnp.float32),
        pltpu.VMEM((D, CW), jnp.float32),
        pltpu.VMEM((D * CW,), jnp.float32),
        pltpu.VMEM((D * CW,), jnp.float32),
        pltpu.SemaphoreType.DMA,
        pltpu.SemaphoreType.DMA,
        pltpu.SemaphoreType.DMA,
        pltpu.SemaphoreType.DMA,
    ],
    compiler_params=pltpu.CompilerParams(needs_layout_passes=False),
)
def _sc_relayout(ut_hbm, it_hbm, uo_hbm, io_hbm,
                 sb0, sb1, ob0, ob1, is0, is1, os0, os1):
    wid = lax.axis_index("s") * NC + lax.axis_index("c")
    nfull = NCW // NW
    rem = NCW - nfull * NW

    srcs = (ut_hbm, it_hbm)
    dsts = (uo_hbm, io_hbm)
    sb = (sb0, sb1)
    ob = (ob0, ob1)
    isem = (is0, is1)
    osem = (os0, os1)

    lane = lax.iota(jnp.int32, 16)

    def issue_in(t, k, p):
        pltpu.async_copy(
            srcs[t].at[:, pl.ds((k * NW + wid) * CW, CW)],
            sb[p], isem[p])

    def scat(p):
        def sbody(g, carry):
            l = lane + g * 16
            fb = lax.shift_right_logical(l, 2) * 128 + (l & 3) * D
            for d in range(D):
                plsc.store_scatter(
                    ob[p], [fb + d], sb[p][d, pl.ds(g * 16, 16)])
            return carry

        lax.fori_loop(0, CW // 16, sbody, 0)

    def wait_in(t, p):
        pltpu.make_async_copy(
            srcs[t].at[:, pl.ds(0, CW)], sb[p], isem[p]).wait()

    def drain_out(t, p):
        pltpu.make_async_copy(
            dsts[t].at[pl.ds(0, CW * D)], ob[p], osem[p]).wait()

    def issue_out(t, k, p):
        pltpu.async_copy(
            ob[p], dsts[t].at[pl.ds((k * NW + wid) * CW * D, CW * D)],
            osem[p])

    def step(t, k, par, last):
        @pl.when(k + 1 < nfull)
        def _():
            issue_in(t, k + 1, 1 - par)
        wait_in(t, par)
        @pl.when(k >= 2)
        def _():
            drain_out(t, par)
        scat(par)
        issue_out(t, k, par)

    for t in (0, 1):
        issue_in(t, 0, 0)

        def kbody(kk, carry):
            for par in (0, 1):
                step(t, kk * 2 + par, par, False)
            return carry

        lax.fori_loop(0, nfull // 2, kbody, 0)
        if nfull % 2:
            step(t, nfull - 1, 0, True)
        for par in (0, 1):
            drain_out(t, par)

    @pl.when(wid < rem)
    def _():
        for t in (0, 1):
            issue_in(t, nfull, 0)
            wait_in(t, 0)
            scat(0)
            issue_out(t, nfull, 0)
            drain_out(t, 0)


@functools.partial(
    pl.kernel,
    mesh=_mesh,
    out_type=jax.ShapeDtypeStruct((B,), jnp.float32),
    scratch_types=[
        pltpu.VMEM((BPW,), jnp.int32),
        pltpu.VMEM((2 * BPW,), jnp.int32),
        pltpu.VMEM((BPW,), jnp.int32),
        pltpu.VMEM((BPW,), jnp.int32),
        pltpu.VMEM((BPW,), jnp.int32),
        pltpu.VMEM((2, CHUNK, 128), jnp.float32),
        pltpu.VMEM((2, CHUNK, 128), jnp.float32),
        pltpu.VMEM((2, CHUNK, 128), jnp.float32),
        pltpu.VMEM((16 * BPW,), jnp.float32),
        pltpu.VMEM((BPW,), jnp.float32),
        pltpu.VMEM((TAIL * D,), jnp.float32),
        pltpu.VMEM((TAIL * D,), jnp.float32),
        pltpu.SemaphoreType.DMA,
        pltpu.SemaphoreType.DMA,
        pltpu.SemaphoreType.DMA,
        pltpu.SemaphoreType.DMA,
        pltpu.SemaphoreType.DMA,
        pltpu.SemaphoreType.DMA,
    ],
    compiler_params=pltpu.CompilerParams(needs_layout_passes=False),
)
def _sc_dots(users_hbm, item_idx_hbm, ut_hbm, it_hbm, tu_hbm, ti_hbm,
             out_hbm,
             ui_v, ii_v, su_v, sp_v, sn_v, u_v, p_v, n_v, t_v, d_v,
             tu_v, ti_v,
             su0, su1, sp0, sp1, sn0, sn1):
    wid = lax.axis_index("s") * NC + lax.axis_index("c")
    base = wid * BPW
    pltpu.sync_copy(users_hbm.at[pl.ds(base, BPW)], ui_v)
    pltpu.sync_copy(item_idx_hbm.at[pl.ds(2 * base, 2 * BPW)], ii_v)
    pltpu.sync_copy(tu_hbm, tu_v)
    pltpu.sync_copy(ti_hbm, ti_v)

    def shift_body(k, carry):
        su_v[pl.ds(k * 16, 16)] = lax.shift_right_logical(
            ui_v[pl.ds(k * 16, 16)], 2)
        sp_v[pl.ds(k * 16, 16)] = lax.shift_right_logical(
            ii_v[pl.ds(k * 16, 16)], 2)
        sn_v[pl.ds(k * 16, 16)] = lax.shift_right_logical(
            ii_v[pl.ds(BPW + k * 16, 16)], 2)
        return carry

    lax.fori_loop(0, BPW // 16, shift_body, 0)

    sems = ((su0, sp0, sn0), (su1, sp1, sn1))

    def issue(c):
        pr = c % 2
        cu = pltpu.async_copy(
            ut_hbm.at[su_v.at[pl.ds(c * CHUNK, CHUNK)]], u_v.at[pr],
            sems[pr][0])
        cp = pltpu.async_copy(
            it_hbm.at[sp_v.at[pl.ds(c * CHUNK, CHUNK)]], p_v.at[pr],
            sems[pr][1])
        cn = pltpu.async_copy(
            it_hbm.at[sn_v.at[pl.ds(c * CHUNK, CHUNK)]], n_v.at[pr],
            sems[pr][2])
        return cu, cp, cn

    lane = lax.iota(jnp.int32, 16)
    lane_off = lane * BPW

    def compute_chunk(c):
        pr = c % 2

        def row_body(r, carry):
            blk = r * 16
            gb = c * CHUNK + blk
            urawv = ui_v[pl.ds(gb, 16)]
            prawv = ii_v[pl.ds(gb, 16)]
            nrawv = ii_v[pl.ds(BPW + gb, 16)]
            quv = (urawv & 3) * D
            qpv = (prawv & 3) * D
            qnv = (nrawv & 3) * D
            one = jnp.ones((16,), jnp.float32)
            zero = jnp.zeros((16,), jnp.float32)
            tuo = jnp.maximum(urawv - TSTART, 0) * D
            tpo = jnp.maximum(prawv - TSTART, 0) * D
            tno = jnp.maximum(nrawv - TSTART, 0) * D
            muv = jnp.where(urawv >= TSTART, one, zero)
            mpv = jnp.where(prawv >= TSTART, one, zero)
            mnv = jnp.where(nrawv >= TSTART, one, zero)
            for j in range(16):
                i = blk + j
                qu = pl.multiple_of(quv[j], D)
                qp = pl.multiple_of(qpv[j], D)
                qn = pl.multiple_of(qnv[j], D)
                ou = pl.multiple_of(tuo[j], D)
                op = pl.multiple_of(tpo[j], D)
                on = pl.multiple_of(tno[j], D)
                u0 = u_v[pr, i, pl.ds(qu, 16)]
                u1 = u_v[pr, i, pl.ds(qu + 16, 16)]
                p0 = p_v[pr, i, pl.ds(qp, 16)]
                p1 = p_v[pr, i, pl.ds(qp + 16, 16)]
                n0 = n_v[pr, i, pl.ds(qn, 16)]
                n1 = n_v[pr, i, pl.ds(qn + 16, 16)]
                mu, mp, mn = muv[j], mpv[j], mnv[j]
                u0 = u0 + mu * (tu_v[pl.ds(ou, 16)] - u0)
                u1 = u1 + mu * (tu_v[pl.ds(ou + 16, 16)] - u1)
                p0 = p0 + mp * (ti_v[pl.ds(op, 16)] - p0)
                p1 = p1 + mp * (ti_v[pl.ds(op + 16, 16)] - p1)
                n0 = n0 + mn * (ti_v[pl.ds(on, 16)] - n0)
                n1 = n1 + mn * (ti_v[pl.ds(on + 16, 16)] - n1)
                s = u0 * (p0 - n0) + u1 * (p1 - n1)
                plsc.store_scatter(t_v, [lane_off + (gb + j)], s)
            return carry

        lax.fori_loop(0, CHUNK // 16, row_body, 0)

    pending = issue(0)
    for c in range(NCH):
        nxt = issue(c + 1) if c + 1 < NCH else None
        for cp in pending:
            cp.wait()
        compute_chunk(c)
        pending = nxt

    def col_body(cb, carry):
        acc = t_v[pl.ds(cb * 16, 16)]
        for k in range(1, 16):
            acc = acc + t_v[pl.ds(k * BPW + cb * 16, 16)]
        d_v[pl.ds(cb * 16, 16)] = acc
        return carry

    lax.fori_loop(0, BPW // 16, col_body, 0)
    pltpu.sync_copy(d_v, out_hbm.at[pl.ds(base, BPW)])


_INV_LN2 = 1.0 / math.log(2.0)


def _loss_body(x_ref, o_ref):
    x = x_ref[...]
    t = -x
    sp = jnp.maximum(t, 0.0) + jnp.log1p(jnp.exp(-jnp.abs(t)))
    o_ref[0, 0] = jnp.sum(sp) * _INV_LN2


_loss_call = pl.pallas_call(
    _loss_body,
    out_shape=jax.ShapeDtypeStruct((1, 1), jnp.float32),
    out_specs=pl.BlockSpec(memory_space=pltpu.SMEM),
)


@jax.jit
def kernel(users, pos_items, neg_items, user_table, item_table):
    users = users.astype(jnp.int32)
    pos_items = pos_items.astype(jnp.int32)
    neg_items = neg_items.astype(jnp.int32)
    item_idx = jnp.concatenate(
        [pos_items.reshape(NW, BPW), neg_items.reshape(NW, BPW)], axis=1
    ).reshape(2 * B)
    ut, it = _sc_relayout(user_table.T, item_table.T)
    tu = user_table[TSTART:, :].reshape(TAIL * D)
    ti = item_table[TSTART:, :].reshape(TAIL * D)
    d = _sc_dots(users, item_idx,
                 ut.reshape(RM_ROWS, 128), it.reshape(RM_ROWS, 128),
                 tu, ti)
    loss = _loss_call(d.reshape(128, 128))
    return loss[0, 0]


# speedup vs baseline: 1.0001x; 1.0001x over previous
import functools
import math

import jax
import jax.numpy as jnp
from jax import lax
from jax.experimental import pallas as pl
from jax.experimental.pallas import tpu as pltpu
from jax.experimental.pallas import tpu_sc as plsc

B = 16384
D = 32
NC = 2
NS = 16
NW = NC * NS
BPW = B // NW
CHUNK = 128
NCH = BPW // CHUNK

NROWS = 1000000
NTC = NROWS // 128
TAIL = NROWS - NTC * 128
TSTART = NTC * 128
CW = 512
NCW = NTC * 128 // CW
RMV = NTC * D
OBP = 138
RM_ROWS = NROWS // 4

_mesh = plsc.VectorSubcoreMesh(core_axis_name="c", subcore_axis_name="s")


@functools.partial(
    pl.kernel,
    mesh=_mesh,
    out_type=(
        jax.ShapeDtypeStruct((RMV, 128), jnp.float32),
        jax.ShapeDtypeStruct((RMV, 128), jnp.float32),
    ),
    scratch_types=[
        pltpu.VMEM((D, CW), jnp.float32),
        pltpu.VMEM((D, CW), jnp.float32),
        pltpu.VMEM((128, OBP), jnp.float32),
        pltpu.VMEM((128, OBP), jnp.float32),
        pltpu.SemaphoreType.DMA,
        pltpu.SemaphoreType.DMA,
        pltpu.SemaphoreType.DMA,
        pltpu.SemaphoreType.DMA,
    ],
    compiler_params=pltpu.CompilerParams(needs_layout_passes=False),
)
def _sc_relayout(ut_hbm, it_hbm, uo_hbm, io_hbm,
                 sb0, sb1, ob0, ob1, is0, is1, os0, os1):
    wid = lax.axis_index("s") * NC + lax.axis_index("c")
    nfull = NCW // NW
    rem = NCW - nfull * NW

    srcs = (ut_hbm, it_hbm)
    dsts = (uo_hbm, io_hbm)
    sb = (sb0, sb1)
    ob = (ob0, ob1)
    isem = (is0, is1)
    osem = (os0, os1)

    lane = lax.iota(jnp.int32, 16)

    def issue_in(t, k, p):
        pltpu.async_copy(
            srcs[t].at[:, pl.ds((k * NW + wid) * CW, CW)],
            sb[p], isem[p])

    def scat(p):
        def sbody(g, carry):
            l = lane + g * 16
            rowv = lax.shift_right_logical(l, 2)
            colb = (l & 3) * D
            for d in range(D):
                plsc.store_scatter(
                    ob[p], [rowv, colb + d], sb[p][d, pl.ds(g * 16, 16)])
            return carry

        lax.fori_loop(0, CW // 16, sbody, 0)

    def wait_in(t, p):
        pltpu.make_async_copy(
            srcs[t].at[:, pl.ds(0, CW)], sb[p], isem[p]).wait()

    def drain_out(t, p):
        pltpu.make_async_copy(
            dsts[t].at[pl.ds(0, 128), :], ob[p].at[:, pl.ds(0, 128)],
            osem[p]).wait()

    def issue_out(t, k, p):
        pltpu.async_copy(
            ob[p].at[:, pl.ds(0, 128)],
            dsts[t].at[pl.ds((k * NW + wid) * 128, 128), :],
            osem[p])

    def step(t, k, par, last):
        @pl.when(k + 1 < nfull)
        def _():
            issue_in(t, k + 1, 1 - par)
        wait_in(t, par)
        @pl.when(k >= 2)
        def _():
            drain_out(t, par)
        scat(par)
        issue_out(t, k, par)

    for t in (0, 1):
        issue_in(t, 0, 0)

        def kbody(kk, carry):
            for par in (0, 1):
                step(t, kk * 2 + par, par, False)
            return carry

        lax.fori_loop(0, nfull // 2, kbody, 0)
        if nfull % 2:
            step(t, nfull - 1, 0, True)
        for par in (0, 1):
            drain_out(t, par)

    @pl.when(wid < rem)
    def _():
        for t in (0, 1):
            issue_in(t, nfull, 0)
            wait_in(t, 0)
            scat(0)
            issue_out(t, nfull, 0)
            drain_out(t, 0)


@functools.partial(
    pl.kernel,
    mesh=_mesh,
    out_type=jax.ShapeDtypeStruct((B,), jnp.float32),
    scratch_types=[
        pltpu.VMEM((BPW,), jnp.int32),
        pltpu.VMEM((2 * BPW,), jnp.int32),
        pltpu.VMEM((BPW,), jnp.int32),
        pltpu.VMEM((BPW,), jnp.int32),
        pltpu.VMEM((BPW,), jnp.int32),
        pltpu.VMEM((2, CHUNK, 128), jnp.float32),
        pltpu.VMEM((2, CHUNK, 128), jnp.float32),
        pltpu.VMEM((2, CHUNK, 128), jnp.float32),
        pltpu.VMEM((16 * BPW,), jnp.float32),
        pltpu.VMEM((BPW,), jnp.float32),
        pltpu.VMEM((TAIL * D,), jnp.float32),
        pltpu.VMEM((TAIL * D,), jnp.float32),
        pltpu.SemaphoreType.DMA,
        pltpu.SemaphoreType.DMA,
        pltpu.SemaphoreType.DMA,
        pltpu.SemaphoreType.DMA,
        pltpu.SemaphoreType.DMA,
        pltpu.SemaphoreType.DMA,
    ],
    compiler_params=pltpu.CompilerParams(needs_layout_passes=False),
)
def _sc_dots(users_hbm, item_idx_hbm, ut_hbm, it_hbm, tu_hbm, ti_hbm,
             out_hbm,
             ui_v, ii_v, su_v, sp_v, sn_v, u_v, p_v, n_v, t_v, d_v,
             tu_v, ti_v,
             su0, su1, sp0, sp1, sn0, sn1):
    wid = lax.axis_index("s") * NC + lax.axis_index("c")
    base = wid * BPW
    pltpu.sync_copy(users_hbm.at[pl.ds(base, BPW)], ui_v)
    pltpu.sync_copy(item_idx_hbm.at[pl.ds(2 * base, 2 * BPW)], ii_v)
    pltpu.sync_copy(tu_hbm, tu_v)
    pltpu.sync_copy(ti_hbm, ti_v)

    def shift_body(k, carry):
        mx = jnp.full((16,), RMV - 1, jnp.int32)
        su_v[pl.ds(k * 16, 16)] = jnp.minimum(lax.shift_right_logical(
            ui_v[pl.ds(k * 16, 16)], 2), mx)
        sp_v[pl.ds(k * 16, 16)] = jnp.minimum(lax.shift_right_logical(
            ii_v[pl.ds(k * 16, 16)], 2), mx)
        sn_v[pl.ds(k * 16, 16)] = jnp.minimum(lax.shift_right_logical(
            ii_v[pl.ds(BPW + k * 16, 16)], 2), mx)
        return carry

    lax.fori_loop(0, BPW // 16, shift_body, 0)

    sems = ((su0, sp0, sn0), (su1, sp1, sn1))

    def issue(c):
        pr = c % 2
        cu = pltpu.async_copy(
            ut_hbm.at[su_v.at[pl.ds(c * CHUNK, CHUNK)]], u_v.at[pr],
            sems[pr][0])
        cp = pltpu.async_copy(
            it_hbm.at[sp_v.at[pl.ds(c * CHUNK, CHUNK)]], p_v.at[pr],
            sems[pr][1])
        cn = pltpu.async_copy(
            it_hbm.at[sn_v.at[pl.ds(c * CHUNK, CHUNK)]], n_v.at[pr],
            sems[pr][2])
        return cu, cp, cn

    lane = lax.iota(jnp.int32, 16)
    lane_off = lane * BPW

    def compute_chunk(c):
        pr = c % 2

        def row_body(r, carry):
            blk = r * 16
            gb = c * CHUNK + blk
            urawv = ui_v[pl.ds(gb, 16)]
            prawv = ii_v[pl.ds(gb, 16)]
            nrawv = ii_v[pl.ds(BPW + gb, 16)]
            quv = (urawv & 3) * D
            qpv = (prawv & 3) * D
            qnv = (nrawv & 3) * D
            one = jnp.ones((16,), jnp.float32)
            zero = jnp.zeros((16,), jnp.float32)
            tuo = jnp.maximum(urawv - TSTART, 0) * D
            tpo = jnp.maximum(prawv - TSTART, 0) * D
            tno = jnp.maximum(nrawv - TSTART, 0) * D
            muv = jnp.where(urawv >= TSTART, one, zero)
            mpv = jnp.where(prawv >= TSTART, one, zero)
            mnv = jnp.where(nrawv >= TSTART, one, zero)
            for j in range(16):
                i = blk + j
                qu = pl.multiple_of(quv[j], D)
                qp = pl.multiple_of(qpv[j], D)
                qn = pl.multiple_of(qnv[j], D)
                ou = pl.multiple_of(tuo[j], D)
                op = pl.multiple_of(tpo[j], D)
                on = pl.multiple_of(tno[j], D)
                u0 = u_v[pr, i, pl.ds(qu, 16)]
                u1 = u_v[pr, i, pl.ds(qu + 16, 16)]
                p0 = p_v[pr, i, pl.ds(qp, 16)]
                p1 = p_v[pr, i, pl.ds(qp + 16, 16)]
                n0 = n_v[pr, i, pl.ds(qn, 16)]
                n1 = n_v[pr, i, pl.ds(qn + 16, 16)]
                mu, mp, mn = muv[j], mpv[j], mnv[j]
                u0 = u0 + mu * (tu_v[pl.ds(ou, 16)] - u0)
                u1 = u1 + mu * (tu_v[pl.ds(ou + 16, 16)] - u1)
                p0 = p0 + mp * (ti_v[pl.ds(op, 16)] - p0)
                p1 = p1 + mp * (ti_v[pl.ds(op + 16, 16)] - p1)
                n0 = n0 + mn * (ti_v[pl.ds(on, 16)] - n0)
                n1 = n1 + mn * (ti_v[pl.ds(on + 16, 16)] - n1)
                s = u0 * (p0 - n0) + u1 * (p1 - n1)
                plsc.store_scatter(t_v, [lane_off + (gb + j)], s)
            return carry

        lax.fori_loop(0, CHUNK // 16, row_body, 0)

    pending = issue(0)
    for c in range(NCH):
        nxt = issue(c + 1) if c + 1 < NCH else None
        for cp in pending:
            cp.wait()
        compute_chunk(c)
        pending = nxt

    def col_body(cb, carry):
        acc = t_v[pl.ds(cb * 16, 16)]
        for k in range(1, 16):
            acc = acc + t_v[pl.ds(k * BPW + cb * 16, 16)]
        d_v[pl.ds(cb * 16, 16)] = acc
        return carry

    lax.fori_loop(0, BPW // 16, col_body, 0)
    pltpu.sync_copy(d_v, out_hbm.at[pl.ds(base, BPW)])


_INV_LN2 = 1.0 / math.log(2.0)


def _loss_body(x_ref, o_ref):
    x = x_ref[...]
    t = -x
    sp = jnp.maximum(t, 0.0) + jnp.log1p(jnp.exp(-jnp.abs(t)))
    o_ref[0, 0] = jnp.sum(sp) * _INV_LN2


_loss_call = pl.pallas_call(
    _loss_body,
    out_shape=jax.ShapeDtypeStruct((1, 1), jnp.float32),
    out_specs=pl.BlockSpec(memory_space=pltpu.SMEM),
)


@jax.jit
def kernel(users, pos_items, neg_items, user_table, item_table):
    users = users.astype(jnp.int32)
    pos_items = pos_items.astype(jnp.int32)
    neg_items = neg_items.astype(jnp.int32)
    item_idx = jnp.concatenate(
        [pos_items.reshape(NW, BPW), neg_items.reshape(NW, BPW)], axis=1
    ).reshape(2 * B)
    ut, it = _sc_relayout(user_table.T, item_table.T)
    tu = user_table[TSTART:, :].reshape(TAIL * D)
    ti = item_table[TSTART:, :].reshape(TAIL * D)
    d = _sc_dots(users, item_idx, ut, it, tu, ti)
    loss = _loss_call(d.reshape(128, 128))
    return loss[0, 0]


# speedup vs baseline: 1.4322x; 1.4321x over previous
import functools
import math

import jax
import jax.numpy as jnp
from jax import lax
from jax.experimental import pallas as pl
from jax.experimental.pallas import tpu as pltpu
from jax.experimental.pallas import tpu_sc as plsc

B = 16384
D = 32
NC = 2
NS = 16
NW = NC * NS
BPW = B // NW
CHUNK = 128
NCH = BPW // CHUNK

_mesh = plsc.VectorSubcoreMesh(core_axis_name="c", subcore_axis_name="s")


@functools.partial(
    pl.kernel,
    mesh=_mesh,
    out_type=jax.ShapeDtypeStruct((B,), jnp.float32),
    scratch_types=[
        pltpu.VMEM((BPW,), jnp.int32),
        pltpu.VMEM((2 * BPW,), jnp.int32),
        pltpu.VMEM((BPW,), jnp.int32),
        pltpu.VMEM((BPW,), jnp.int32),
        pltpu.VMEM((BPW,), jnp.int32),
        pltpu.VMEM((2, CHUNK, 128), jnp.float32),
        pltpu.VMEM((2, CHUNK, 128), jnp.float32),
        pltpu.VMEM((2, CHUNK, 128), jnp.float32),
        pltpu.VMEM((16 * BPW,), jnp.float32),
        pltpu.VMEM((BPW,), jnp.float32),
        pltpu.SemaphoreType.DMA,
        pltpu.SemaphoreType.DMA,
        pltpu.SemaphoreType.DMA,
        pltpu.SemaphoreType.DMA,
        pltpu.SemaphoreType.DMA,
        pltpu.SemaphoreType.DMA,
    ],
    compiler_params=pltpu.CompilerParams(needs_layout_passes=False),
)
def _sc_dots(users_hbm, item_idx_hbm, ut_hbm, it_hbm, out_hbm,
             ui_v, ii_v, su_v, sp_v, sn_v, u_v, p_v, n_v, t_v, d_v,
             su0, su1, sp0, sp1, sn0, sn1):
    wid = lax.axis_index("s") * NC + lax.axis_index("c")
    base = wid * BPW
    pltpu.sync_copy(users_hbm.at[pl.ds(base, BPW)], ui_v)
    pltpu.sync_copy(item_idx_hbm.at[pl.ds(2 * base, 2 * BPW)], ii_v)

    def shift_body(k, carry):
        su_v[pl.ds(k * 16, 16)] = lax.shift_right_logical(
            ui_v[pl.ds(k * 16, 16)], 2)
        sp_v[pl.ds(k * 16, 16)] = lax.shift_right_logical(
            ii_v[pl.ds(k * 16, 16)], 2)
        sn_v[pl.ds(k * 16, 16)] = lax.shift_right_logical(
            ii_v[pl.ds(BPW + k * 16, 16)], 2)
        return carry

    lax.fori_loop(0, BPW // 16, shift_body, 0)

    sems = ((su0, sp0, sn0), (su1, sp1, sn1))

    def issue(c):
        pr = c % 2
        cu = pltpu.async_copy(
            ut_hbm.at[su_v.at[pl.ds(c * CHUNK, CHUNK)]], u_v.at[pr],
            sems[pr][0])
        cp = pltpu.async_copy(
            it_hbm.at[sp_v.at[pl.ds(c * CHUNK, CHUNK)]], p_v.at[pr],
            sems[pr][1])
        cn = pltpu.async_copy(
            it_hbm.at[sn_v.at[pl.ds(c * CHUNK, CHUNK)]], n_v.at[pr],
            sems[pr][2])
        return cu, cp, cn

    lane = lax.iota(jnp.int32, 16)
    lane_off = lane * BPW

    def compute_chunk(c):
        pr = c % 2

        def row_body(r, carry):
            blk = r * 16
            gb = c * CHUNK + blk
            quv = (ui_v[pl.ds(gb, 16)] & 3) * D
            qpv = (ii_v[pl.ds(gb, 16)] & 3) * D
            qnv = (ii_v[pl.ds(BPW + gb, 16)] & 3) * D
            for j in range(16):
                i = blk + j
                qu = pl.multiple_of(quv[j], D)
                qp = pl.multiple_of(qpv[j], D)
                qn = pl.multiple_of(qnv[j], D)
                u0 = u_v[pr, i, pl.ds(qu, 16)]
                u1 = u_v[pr, i, pl.ds(qu + 16, 16)]
                p0 = p_v[pr, i, pl.ds(qp, 16)]
                p1 = p_v[pr, i, pl.ds(qp + 16, 16)]
                n0 = n_v[pr, i, pl.ds(qn, 16)]
                n1 = n_v[pr, i, pl.ds(qn + 16, 16)]
                s = u0 * (p0 - n0) + u1 * (p1 - n1)
                plsc.store_scatter(t_v, [lane_off + (gb + j)], s)
            return carry

        lax.fori_loop(0, CHUNK // 16, row_body, 0)

    pending = issue(0)
    for c in range(NCH):
        nxt = issue(c + 1) if c + 1 < NCH else None
        for cp in pending:
            cp.wait()
        compute_chunk(c)
        pending = nxt

    def col_body(cb, carry):
        acc = t_v[pl.ds(cb * 16, 16)]
        for k in range(1, 16):
            acc = acc + t_v[pl.ds(k * BPW + cb * 16, 16)]
        d_v[pl.ds(cb * 16, 16)] = acc
        return carry

    lax.fori_loop(0, BPW // 16, col_body, 0)
    pltpu.sync_copy(d_v, out_hbm.at[pl.ds(base, BPW)])


_INV_LN2 = 1.0 / math.log(2.0)


def _loss_body(x_ref, o_ref):
    x = x_ref[...]
    t = -x
    sp = jnp.maximum(t, 0.0) + jnp.log1p(jnp.exp(-jnp.abs(t)))
    o_ref[0, 0] = jnp.sum(sp) * _INV_LN2


_loss_call = pl.pallas_call(
    _loss_body,
    out_shape=jax.ShapeDtypeStruct((1, 1), jnp.float32),
    out_specs=pl.BlockSpec(memory_space=pltpu.SMEM),
)


@jax.jit
def kernel(users, pos_items, neg_items, user_table, item_table):
    users = users.astype(jnp.int32)
    pos_items = pos_items.astype(jnp.int32)
    neg_items = neg_items.astype(jnp.int32)
    item_idx = jnp.concatenate(
        [pos_items.reshape(NW, BPW), neg_items.reshape(NW, BPW)], axis=1
    ).reshape(2 * B)
    ut = user_table.reshape(-1, 128)
    it = item_table.reshape(-1, 128)
    d = _sc_dots(users, item_idx, ut, it)
    loss = _loss_call(d.reshape(128, 128))
    return loss[0, 0]
